# Initial kernel scaffold; baseline (speedup 1.0000x reference)
import jax
import jax.numpy as jnp
from jax.experimental import pallas as pl


def kernel(states, action, We1, be1, We2, be2, ge, gb, We3, be3, Wn1, bn1, Wn2, bn2, gn, gnb, Wn3, bn3):
    raise NotImplementedError("write your pallas kernel here")


# speedup vs baseline: 20.5172x; 20.5172x over previous
import functools

import jax
import jax.numpy as jnp
from jax.experimental import pallas as pl

B, K, D, H, A = 512, 32, 64, 64, 4
G = 8


def _ln(x, g, b):
    m = jnp.mean(x, axis=-1, keepdims=True)
    xc = x - m
    v = jnp.mean(xc * xc, axis=-1, keepdims=True)
    return xc * jax.lax.rsqrt(v + 1e-5) * g + b


def _gnn_kernel(node_ref, av_ref, we1a_ref, we1b_ref, be1_ref, we2_ref,
                be2_ref, ge_ref, gb_ref, we3_ref, be3_ref, wn1a_ref,
                wn1b_ref, wn1c_ref, bn1_ref, wn2_ref, bn2_ref, gn_ref,
                gnb_ref, wn3_ref, bn3_ref, out_ref):
    node = node_ref[...]
    av = av_ref[...]

    p = jnp.dot(node, we1a_ref[...], preferred_element_type=jnp.float32)
    q = jnp.dot(node, we1b_ref[...], preferred_element_type=jnp.float32)
    p = p + be1_ref[...]

    p4 = p.reshape(G, K, 1, H)
    q4 = q.reshape(G, 1, K, H)
    h1 = jax.nn.relu(p4 + q4).reshape(G * K * K, H)

    h2 = jnp.dot(h1, we2_ref[...], preferred_element_type=jnp.float32)
    h2 = jax.nn.relu(_ln(h2 + be2_ref[...], ge_ref[...], gb_ref[...]))
    ea = jnp.dot(h2, we3_ref[...], preferred_element_type=jnp.float32)
    ea = ea + be3_ref[...]

    ea4 = ea.reshape(G * K, K, H)
    i_idx = jax.lax.broadcasted_iota(jnp.int32, (G * K, K, H), 0) % K
    j_idx = jax.lax.broadcasted_iota(jnp.int32, (G * K, K, H), 1)
    ea4 = jnp.where(i_idx == j_idx, 0.0, ea4)
    agg = jnp.sum(ea4, axis=1)

    z = (jnp.dot(node, wn1a_ref[...], preferred_element_type=jnp.float32)
         + jnp.dot(av, wn1b_ref[...], preferred_element_type=jnp.float32)
         + jnp.dot(agg, wn1c_ref[...], preferred_element_type=jnp.float32)
         + bn1_ref[...])
    z = jax.nn.relu(z)
    z2 = jnp.dot(z, wn2_ref[...], preferred_element_type=jnp.float32)
    z2 = jax.nn.relu(_ln(z2 + bn2_ref[...], gn_ref[...], gnb_ref[...]))
    out = jnp.dot(z2, wn3_ref[...], preferred_element_type=jnp.float32)
    out_ref[...] = out + bn3_ref[...]


@jax.jit
def kernel(states, action, We1, be1, We2, be2, ge, gb, We3, be3,
           Wn1, bn1, Wn2, bn2, gn, gnb, Wn3, bn3):
    node = states.reshape(B * K, D)
    av = action.reshape(B * K, A)

    we1a = We1[:, :D].T
    we1b = We1[:, D:].T
    wn1a = Wn1[:, :D].T
    wn1b = Wn1[:, D:D + A].T
    wn1c = Wn1[:, D + A:].T

    row = lambda v: v.reshape(1, -1)
    weights = [we1a, we1b, row(be1), We2.T, row(be2), row(ge), row(gb),
               We3.T, row(be3), wn1a, wn1b, wn1c, row(bn1), Wn2.T,
               row(bn2), row(gn), row(gnb), Wn3.T, row(bn3)]

    full = lambda a: pl.BlockSpec(a.shape, lambda i: (0,) * a.ndim)
    grid = B // G
    out = pl.pallas_call(
        _gnn_kernel,
        grid=(grid,),
        in_specs=[pl.BlockSpec((G * K, D), lambda i: (i, 0)),
                  pl.BlockSpec((G * K, A), lambda i: (i, 0))]
                 + [full(w) for w in weights],
        out_specs=pl.BlockSpec((G * K, D), lambda i: (i, 0)),
        out_shape=jax.ShapeDtypeStruct((B * K, D), jnp.float32),
    )(node, av, *weights)
    return out.reshape(B, K, D)


# speedup vs baseline: 26.5728x; 1.2951x over previous
import jax
import jax.numpy as jnp
from jax.experimental import pallas as pl

B, K, D, H, A = 512, 32, 64, 64, 4
G = 8
H2 = 2 * H


def _gnn_kernel(node_ref, av_ref, we1a_ref, we1b_ref, be1_ref, we2d_ref,
                be2d_ref, ged_ref, gbd_ref, we3d_ref, be3d_ref, jd_ref,
                wn1a_ref, wn1b_ref, wn1c_ref, bn1_ref, wn2_ref, bn2_ref,
                gn_ref, gnb_ref, wn3_ref, bn3_ref, out_ref):
    node = node_ref[...]
    av = av_ref[...]
    jd = jd_ref[...]

    p = jnp.dot(node, we1a_ref[...], preferred_element_type=jnp.float32)
    q = jnp.dot(node, we1b_ref[...], preferred_element_type=jnp.float32)
    p = p + be1_ref[...]

    pp = jnp.concatenate([p, p], axis=-1).reshape(G, K, 1, H2)
    q3 = q.reshape(G, K, H)
    qp = jnp.concatenate([q3[:, :K // 2, :], q3[:, K // 2:, :]], axis=-1)
    h1 = jax.nn.relu(pp + qp.reshape(G, 1, K // 2, H2))
    h1 = h1.reshape(G * K * K // 2, H2)

    z = jnp.dot(h1, we2d_ref[...], preferred_element_type=jnp.float32)
    z = z + be2d_ref[...]
    m = jnp.dot(z, jd, preferred_element_type=jnp.float32)
    zc = z - m
    v = jnp.dot(zc * zc, jd, preferred_element_type=jnp.float32)
    h2 = jax.nn.relu(zc * jax.lax.rsqrt(v + 1e-5) * ged_ref[...]
                     + gbd_ref[...])

    ea = jnp.dot(h2, we3d_ref[...], preferred_element_type=jnp.float32)
    ea = ea + be3d_ref[...]

    ea3 = ea.reshape(G * K, K // 2, H2)
    i_idx = jax.lax.broadcasted_iota(jnp.int32, (G * K, K // 2, H2), 0) % K
    c_idx = jax.lax.broadcasted_iota(jnp.int32, (G * K, K // 2, H2), 1)
    l_idx = jax.lax.broadcasted_iota(jnp.int32, (G * K, K // 2, H2), 2)
    j_idx = c_idx + (K // 2) * (l_idx >= H).astype(jnp.int32)
    ea3 = jnp.where(i_idx == j_idx, 0.0, ea3)
    agg2 = jnp.sum(ea3, axis=1)
    agg = agg2[:, :H] + agg2[:, H:]

    z = (jnp.dot(node, wn1a_ref[...], preferred_element_type=jnp.float32)
         + jnp.dot(av, wn1b_ref[...], preferred_element_type=jnp.float32)
         + jnp.dot(agg, wn1c_ref[...], preferred_element_type=jnp.float32)
         + bn1_ref[...])
    z = jax.nn.relu(z)
    z2 = jnp.dot(z, wn2_ref[...], preferred_element_type=jnp.float32)
    z2 = z2 + bn2_ref[...]
    m = jnp.mean(z2, axis=-1, keepdims=True)
    zc = z2 - m
    v = jnp.mean(zc * zc, axis=-1, keepdims=True)
    z2 = jax.nn.relu(zc * jax.lax.rsqrt(v + 1e-5) * gn_ref[...]
                     + gnb_ref[...])
    out = jnp.dot(z2, wn3_ref[...], preferred_element_type=jnp.float32)
    out_ref[...] = out + bn3_ref[...]


def _blockdiag(w):
    z = jnp.zeros_like(w)
    return jnp.concatenate([jnp.concatenate([w, z], 1),
                            jnp.concatenate([z, w], 1)], 0)


@jax.jit
def kernel(states, action, We1, be1, We2, be2, ge, gb, We3, be3,
           Wn1, bn1, Wn2, bn2, gn, gnb, Wn3, bn3):
    node = states.reshape(B * K, D)
    av = action.reshape(B * K, A)

    we1a = We1[:, :D].T
    we1b = We1[:, D:].T
    wn1a = Wn1[:, :D].T
    wn1b = Wn1[:, D:D + A].T
    wn1c = Wn1[:, D + A:].T
    jd = _blockdiag(jnp.full((H, H), 1.0 / H, jnp.float32))

    row = lambda v: v.reshape(1, -1)
    two = lambda v: jnp.concatenate([v, v]).reshape(1, -1)
    weights = [we1a, we1b, row(be1), _blockdiag(We2.T), two(be2), two(ge),
               two(gb), _blockdiag(We3.T), two(be3), jd, wn1a, wn1b, wn1c,
               row(bn1), Wn2.T, row(bn2), row(gn), row(gnb), Wn3.T,
               row(bn3)]

    full = lambda a: pl.BlockSpec(a.shape, lambda i: (0,) * a.ndim)
    out = pl.pallas_call(
        _gnn_kernel,
        grid=(B // G,),
        in_specs=[pl.BlockSpec((G * K, D), lambda i: (i, 0)),
                  pl.BlockSpec((G * K, A), lambda i: (i, 0))]
                 + [full(w) for w in weights],
        out_specs=pl.BlockSpec((G * K, D), lambda i: (i, 0)),
        out_shape=jax.ShapeDtypeStruct((B * K, D), jnp.float32),
    )(node, av, *weights)
    return out.reshape(B, K, D)


# speedup vs baseline: 31.3746x; 1.1807x over previous
import jax
import jax.numpy as jnp
from jax.experimental import pallas as pl

B, K, D, H, A = 512, 32, 64, 64, 4
G = 8
H2 = 2 * H


def _gnn_kernel(node_ref, av_ref, we1a_ref, we1b_ref, be1_ref, wc2_ref,
                bc2_ref, ged_ref, gbd_ref, jd_ref, wn1a_ref, wn1b_ref,
                wagg_ref, bn1f_ref, wn2c_ref, bn2c_ref, gn_ref, gnb_ref,
                wn3_ref, bn3_ref, out_ref):
    node = node_ref[...]
    av = av_ref[...]
    jd = jd_ref[...]

    p = jnp.dot(node, we1a_ref[...], preferred_element_type=jnp.float32)
    q = jnp.dot(node, we1b_ref[...], preferred_element_type=jnp.float32)
    p = p + be1_ref[...]

    pp = jnp.concatenate([p, p], axis=-1).reshape(G, K, 1, H2)
    q3 = q.reshape(G, K, H)
    qp = jnp.concatenate([q3[:, :K // 2, :], q3[:, K // 2:, :]], axis=-1)
    h1 = jax.nn.relu(pp + qp.reshape(G, 1, K // 2, H2))
    h1 = h1.reshape(G * K * K // 2, H2)

    zc = jnp.dot(h1, wc2_ref[...], preferred_element_type=jnp.float32)
    zc = zc + bc2_ref[...]
    v = jnp.dot(zc * zc, jd, preferred_element_type=jnp.float32)
    h2 = jax.nn.relu(zc * jax.lax.rsqrt(v + 1e-5) * ged_ref[...]
                     + gbd_ref[...])

    h23 = h2.reshape(G * K, K // 2, H2)
    i_idx = jax.lax.broadcasted_iota(jnp.int32, (G * K, K // 2, H2), 0) % K
    c_idx = jax.lax.broadcasted_iota(jnp.int32, (G * K, K // 2, H2), 1)
    l_idx = jax.lax.broadcasted_iota(jnp.int32, (G * K, K // 2, H2), 2)
    j_idx = c_idx + (K // 2) * (l_idx >= H).astype(jnp.int32)
    h23 = jnp.where(i_idx == j_idx, 0.0, h23)
    s2 = jnp.sum(h23, axis=1)
    s = s2[:, :H] + s2[:, H:]

    z = (jnp.dot(node, wn1a_ref[...], preferred_element_type=jnp.float32)
         + jnp.dot(av, wn1b_ref[...], preferred_element_type=jnp.float32)
         + jnp.dot(s, wagg_ref[...], preferred_element_type=jnp.float32)
         + bn1f_ref[...])
    z = jax.nn.relu(z)
    zc2 = jnp.dot(z, wn2c_ref[...], preferred_element_type=jnp.float32)
    zc2 = zc2 + bn2c_ref[...]
    v2 = jnp.mean(zc2 * zc2, axis=-1, keepdims=True)
    z2 = jax.nn.relu(zc2 * jax.lax.rsqrt(v2 + 1e-5) * gn_ref[...]
                     + gnb_ref[...])
    out = jnp.dot(z2, wn3_ref[...], preferred_element_type=jnp.float32)
    out_ref[...] = out + bn3_ref[...]


def _blockdiag(w):
    z = jnp.zeros_like(w)
    return jnp.concatenate([jnp.concatenate([w, z], 1),
                            jnp.concatenate([z, w], 1)], 0)


@jax.jit
def kernel(states, action, We1, be1, We2, be2, ge, gb, We3, be3,
           Wn1, bn1, Wn2, bn2, gn, gnb, Wn3, bn3):
    node = states.reshape(B * K, D)
    av = action.reshape(B * K, A)

    we1a = We1[:, :D].T
    we1b = We1[:, D:].T
    wn1a = Wn1[:, :D].T
    wn1b = Wn1[:, D:D + A].T
    wn1c = Wn1[:, D + A:].T
    jd = _blockdiag(jnp.full((H, H), 1.0 / H, jnp.float32))

    we2d = _blockdiag(We2.T)
    be2d = jnp.concatenate([be2, be2])
    wc2 = we2d - we2d @ jd
    bc2 = (be2d - jnp.mean(be2)).reshape(1, -1)
    wn2c_m = Wn2.T - jnp.mean(Wn2.T, axis=1, keepdims=True)
    bn2c = (bn2 - jnp.mean(bn2)).reshape(1, -1)

    wagg = We3.T @ wn1c
    bn1f = (bn1 + (K - 1) * (be3 @ wn1c)).reshape(1, -1)

    row = lambda v: v.reshape(1, -1)
    two = lambda v: jnp.concatenate([v, v]).reshape(1, -1)
    weights = [we1a, we1b, row(be1), wc2, bc2, two(ge), two(gb), jd,
               wn1a, wn1b, wagg, bn1f, wn2c_m, bn2c, row(gn), row(gnb),
               Wn3.T, row(bn3)]

    full = lambda a: pl.BlockSpec(a.shape, lambda i: (0,) * a.ndim)
    out = pl.pallas_call(
        _gnn_kernel,
        grid=(B // G,),
        in_specs=[pl.BlockSpec((G * K, D), lambda i: (i, 0)),
                  pl.BlockSpec((G * K, A), lambda i: (i, 0))]
                 + [full(w) for w in weights],
        out_specs=pl.BlockSpec((G * K, D), lambda i: (i, 0)),
        out_shape=jax.ShapeDtypeStruct((B * K, D), jnp.float32),
    )(node, av, *weights)
    return out.reshape(B, K, D)
